# Initial kernel scaffold; baseline (speedup 1.0000x reference)
import jax
import jax.numpy as jnp
from jax.experimental import pallas as pl


def kernel(tokens, cb0, cb1, cb2, Wq, bq, Wk, bk, Wv, bv, Wo, bo, gamma, beta):
    raise NotImplementedError("write your pallas kernel here")


# speedup vs baseline: 6.5475x; 6.5475x over previous
import functools
import math

import jax
import jax.numpy as jnp
from jax import lax
from jax.experimental import pallas as pl
from jax.experimental.pallas import tpu as pltpu, tpu_sc as plsc

B = 16384
E = 384
H = 4
DH = E // H
K = 1024
S = 3

NC = 2
NS = 16
NW = NC * NS
B_PER_W = B // NW
CH = 128
NCHUNK = B_PER_W // CH


def _sc_gather(t0, t1, t2, cb0, cb1, cb2):
    mesh = plsc.VectorSubcoreMesh(
        core_axis_name="c", subcore_axis_name="s",
        num_cores=NC, num_subcores=NS)

    @functools.partial(
        pl.kernel,
        out_type=jax.ShapeDtypeStruct((S, B, E), jnp.float32),
        mesh=mesh,
        scratch_types=[
            pltpu.VMEM((B_PER_W,), jnp.int32),
            pltpu.VMEM((B_PER_W,), jnp.int32),
            pltpu.VMEM((B_PER_W,), jnp.int32),
            pltpu.VMEM((CH, E), jnp.float32),
            pltpu.VMEM((CH, E), jnp.float32),
            pltpu.SemaphoreType.DMA,
            pltpu.SemaphoreType.DMA,
            pltpu.SemaphoreType.DMA,
            pltpu.SemaphoreType.DMA,
        ],
    )
    def gather_kernel(t0_hbm, t1_hbm, t2_hbm, cb0_hbm, cb1_hbm, cb2_hbm,
                      out_hbm, idx0_v, idx1_v, idx2_v, buf0, buf1,
                      sg0, sg1, sw0, sw1):
        wid = lax.axis_index("s") * NC + lax.axis_index("c")
        base = wid * B_PER_W
        tok_refs = (t0_hbm, t1_hbm, t2_hbm)
        cb_refs = (cb0_hbm, cb1_hbm, cb2_hbm)
        idx_refs = (idx0_v, idx1_v, idx2_v)
        for ti in range(S):
            pltpu.sync_copy(tok_refs[ti].at[pl.ds(base, B_PER_W)],
                            idx_refs[ti])

        bufs = (buf0, buf1)
        gsems = (sg0, sg1)
        wsems = (sw0, sw1)
        chunks = [(ti, c) for ti in range(S) for c in range(NCHUNK)]
        n_total = len(chunks)

        def start_gather(n):
            ti, c = chunks[n]
            return pltpu.async_copy(
                cb_refs[ti].at[idx_refs[ti].at[pl.ds(c * CH, CH)]],
                bufs[n % 2], gsems[n % 2])

        def start_write(n):
            ti, c = chunks[n]
            return pltpu.async_copy(
                bufs[n % 2],
                out_hbm.at[ti, pl.ds(base + c * CH, CH)],
                wsems[n % 2])

        cp_g = start_gather(0)
        cp_w = [None, None]
        for n in range(n_total):
            nxt = None
            if n + 1 < n_total:
                nb = (n + 1) % 2
                if cp_w[nb] is not None:
                    cp_w[nb].wait()
                    cp_w[nb] = None
                nxt = start_gather(n + 1)
            cp_g.wait()
            cp_w[n % 2] = start_write(n)
            cp_g = nxt
        for w in cp_w:
            if w is not None:
                w.wait()

    return gather_kernel(t0, t1, t2, cb0, cb1, cb2)


BB = 512
EPS = 1e-5
SCALE = 1.0 / math.sqrt(DH)


def _tc_body(g_ref, wqkv_ref, bqkv_ref, hm_ref, hmt_ref, wo_ref, bo_ref,
             gamma_ref, beta_ref, out_ref):
    e = g_ref[...].reshape(S * BB, E)
    qkv = jnp.dot(e.astype(jnp.bfloat16), wqkv_ref[...],
                  preferred_element_type=jnp.float32) + bqkv_ref[...]
    q = [qkv[i * BB:(i + 1) * BB, 0:E] for i in range(S)]
    k = [qkv[i * BB:(i + 1) * BB, E:2 * E] for i in range(S)]
    v = [qkv[i * BB:(i + 1) * BB, 2 * E:3 * E] for i in range(S)]

    hm = hm_ref[...]
    hmt = hmt_ref[...]
    s = [[jnp.dot((q[i] * k[j]).astype(jnp.bfloat16), hm,
                  preferred_element_type=jnp.float32) * SCALE
          for j in range(S)] for i in range(S)]

    out_pre = []
    for i in range(S):
        m = jnp.maximum(jnp.maximum(s[i][0], s[i][1]), s[i][2])
        ex = [jnp.exp(s[i][j] - m) for j in range(S)]
        den = ex[0] + ex[1] + ex[2]
        acc = jnp.zeros((BB, E), jnp.float32)
        for j in range(S):
            a = (ex[j] / den).astype(jnp.bfloat16)
            aexp = jnp.dot(a, hmt, preferred_element_type=jnp.float32)
            acc = acc + aexp * v[j]
        out_pre.append(acc)

    op = jnp.concatenate(out_pre, axis=0)
    x = jnp.dot(op.astype(jnp.bfloat16), wo_ref[...],
                preferred_element_type=jnp.float32) + bo_ref[...] + e
    mu = jnp.mean(x, axis=-1, keepdims=True)
    xc = x - mu
    var = jnp.mean(xc * xc, axis=-1, keepdims=True)
    y = xc * lax.rsqrt(var + EPS) * gamma_ref[...] + beta_ref[...]
    out_ref[...] = (y[0:BB] + y[BB:2 * BB] + y[2 * BB:3 * BB]) * (1.0 / 3.0)


def _tc_compute(g, wqkv16, bqkv, hm16, hmt16, wo16, bo2, gamma2, beta2):
    n_blocks = B // BB
    const = lambda b: (0, 0)
    return pl.pallas_call(
        _tc_body,
        grid=(n_blocks,),
        in_specs=[
            pl.BlockSpec((S, BB, E), lambda b: (0, b, 0)),
            pl.BlockSpec((E, 3 * E), const),
            pl.BlockSpec((1, 3 * E), const),
            pl.BlockSpec((E, H), const),
            pl.BlockSpec((H, E), const),
            pl.BlockSpec((E, E), const),
            pl.BlockSpec((1, E), const),
            pl.BlockSpec((1, E), const),
            pl.BlockSpec((1, E), const),
        ],
        out_specs=pl.BlockSpec((BB, E), lambda b: (b, 0)),
        out_shape=jax.ShapeDtypeStruct((B, E), jnp.float32),
    )(g, wqkv16, bqkv, hm16, hmt16, wo16, bo2, gamma2, beta2)


def kernel(tokens, cb0, cb1, cb2, Wq, bq, Wk, bk, Wv, bv, Wo, bo, gamma, beta):
    t0 = tokens[:, 0]
    t1 = tokens[:, 1]
    t2 = tokens[:, 2]
    g = _sc_gather(t0, t1, t2, cb0, cb1, cb2)

    wqkv16 = jnp.concatenate([Wq.T, Wk.T, Wv.T], axis=1).astype(jnp.bfloat16)
    bqkv = jnp.concatenate([bq, bk, bv]).reshape(1, 3 * E)
    head_of = jnp.arange(E, dtype=jnp.int32) // DH
    hm = (head_of[:, None] == jnp.arange(H, dtype=jnp.int32)[None, :])
    hm16 = hm.astype(jnp.bfloat16)
    hmt16 = hm.T.astype(jnp.bfloat16)
    wo16 = Wo.T.astype(jnp.bfloat16)
    return _tc_compute(g, wqkv16, bqkv, hm16, hmt16, wo16,
                       bo.reshape(1, E), gamma.reshape(1, E),
                       beta.reshape(1, E))


# speedup vs baseline: 6.9068x; 1.0549x over previous
import functools
import math

import jax
import jax.numpy as jnp
from jax import lax
from jax.experimental import pallas as pl
from jax.experimental.pallas import tpu as pltpu, tpu_sc as plsc

B = 16384
E = 384
H = 4
DH = E // H
K = 1024
S = 3

NC = 2
NS = 16
NW = NC * NS


def _sc_gather(t0, t1, t2, cb0, cb1, cb2, rows):
    B_PER_W = rows // NW
    CH = min(128, B_PER_W)
    NCHUNK = B_PER_W // CH
    mesh = plsc.VectorSubcoreMesh(
        core_axis_name="c", subcore_axis_name="s",
        num_cores=NC, num_subcores=NS)

    @functools.partial(
        pl.kernel,
        out_type=jax.ShapeDtypeStruct((S, rows, E), jnp.float32),
        mesh=mesh,
        scratch_types=[
            pltpu.VMEM((B_PER_W,), jnp.int32),
            pltpu.VMEM((B_PER_W,), jnp.int32),
            pltpu.VMEM((B_PER_W,), jnp.int32),
            pltpu.VMEM((CH, E), jnp.float32),
            pltpu.VMEM((CH, E), jnp.float32),
            pltpu.SemaphoreType.DMA,
            pltpu.SemaphoreType.DMA,
            pltpu.SemaphoreType.DMA,
            pltpu.SemaphoreType.DMA,
        ],
    )
    def gather_kernel(t0_hbm, t1_hbm, t2_hbm, cb0_hbm, cb1_hbm, cb2_hbm,
                      out_hbm, idx0_v, idx1_v, idx2_v, buf0, buf1,
                      sg0, sg1, sw0, sw1):
        wid = lax.axis_index("s") * NC + lax.axis_index("c")
        base = wid * B_PER_W
        tok_refs = (t0_hbm, t1_hbm, t2_hbm)
        cb_refs = (cb0_hbm, cb1_hbm, cb2_hbm)
        idx_refs = (idx0_v, idx1_v, idx2_v)
        for ti in range(S):
            pltpu.sync_copy(tok_refs[ti].at[pl.ds(base, B_PER_W)],
                            idx_refs[ti])

        bufs = (buf0, buf1)
        gsems = (sg0, sg1)
        wsems = (sw0, sw1)
        chunks = [(ti, c) for ti in range(S) for c in range(NCHUNK)]
        n_total = len(chunks)

        def start_gather(n):
            ti, c = chunks[n]
            return pltpu.async_copy(
                cb_refs[ti].at[idx_refs[ti].at[pl.ds(c * CH, CH)]],
                bufs[n % 2], gsems[n % 2])

        def start_write(n):
            ti, c = chunks[n]
            return pltpu.async_copy(
                bufs[n % 2],
                out_hbm.at[ti, pl.ds(base + c * CH, CH)],
                wsems[n % 2])

        cp_g = start_gather(0)
        cp_w = [None, None]
        for n in range(n_total):
            nxt = None
            if n + 1 < n_total:
                nb = (n + 1) % 2
                if cp_w[nb] is not None:
                    cp_w[nb].wait()
                    cp_w[nb] = None
                nxt = start_gather(n + 1)
            cp_g.wait()
            cp_w[n % 2] = start_write(n)
            cp_g = nxt
        for w in cp_w:
            if w is not None:
                w.wait()

    return gather_kernel(t0, t1, t2, cb0, cb1, cb2)


BB = 512
EPS = 1e-5
SCALE = 1.0 / math.sqrt(DH)


def _tc_body(g_ref, wqkv_ref, bqkv_ref, hm_ref, hmt_ref, wo_ref, bo_ref,
             gamma_ref, beta_ref, out_ref):
    e = g_ref[...].reshape(S * BB, E)
    qkv = jnp.dot(e.astype(jnp.bfloat16), wqkv_ref[...],
                  preferred_element_type=jnp.float32) + bqkv_ref[...]
    q = [qkv[i * BB:(i + 1) * BB, 0:E] for i in range(S)]
    k = [qkv[i * BB:(i + 1) * BB, E:2 * E] for i in range(S)]
    v = [qkv[i * BB:(i + 1) * BB, 2 * E:3 * E] for i in range(S)]

    hm = hm_ref[...]
    hmt = hmt_ref[...]
    s = [[jnp.dot((q[i] * k[j]).astype(jnp.bfloat16), hm,
                  preferred_element_type=jnp.float32) * SCALE
          for j in range(S)] for i in range(S)]

    out_pre = []
    for i in range(S):
        m = jnp.maximum(jnp.maximum(s[i][0], s[i][1]), s[i][2])
        ex = [jnp.exp(s[i][j] - m) for j in range(S)]
        den = ex[0] + ex[1] + ex[2]
        acc = jnp.zeros((BB, E), jnp.float32)
        for j in range(S):
            a = (ex[j] / den).astype(jnp.bfloat16)
            aexp = jnp.dot(a, hmt, preferred_element_type=jnp.float32)
            acc = acc + aexp * v[j]
        out_pre.append(acc)

    op = jnp.concatenate(out_pre, axis=0)
    x = jnp.dot(op.astype(jnp.bfloat16), wo_ref[...],
                preferred_element_type=jnp.float32) + bo_ref[...] + e
    mu = jnp.mean(x, axis=-1, keepdims=True)
    xc = x - mu
    var = jnp.mean(xc * xc, axis=-1, keepdims=True)
    y = xc * lax.rsqrt(var + EPS) * gamma_ref[...] + beta_ref[...]
    out_ref[...] = (y[0:BB] + y[BB:2 * BB] + y[2 * BB:3 * BB]) * (1.0 / 3.0)


def _tc_compute(g, wqkv16, bqkv, hm16, hmt16, wo16, bo2, gamma2, beta2):
    rows = g.shape[1]
    n_blocks = rows // BB
    const = lambda b: (0, 0)
    return pl.pallas_call(
        _tc_body,
        grid=(n_blocks,),
        in_specs=[
            pl.BlockSpec((S, BB, E), lambda b: (0, b, 0)),
            pl.BlockSpec((E, 3 * E), const),
            pl.BlockSpec((1, 3 * E), const),
            pl.BlockSpec((E, H), const),
            pl.BlockSpec((H, E), const),
            pl.BlockSpec((E, E), const),
            pl.BlockSpec((1, E), const),
            pl.BlockSpec((1, E), const),
            pl.BlockSpec((1, E), const),
        ],
        out_specs=pl.BlockSpec((BB, E), lambda b: (b, 0)),
        out_shape=jax.ShapeDtypeStruct((rows, E), jnp.float32),
    )(g, wqkv16, bqkv, hm16, hmt16, wo16, bo2, gamma2, beta2)


NSPLIT = 4
CHUNK = B // NSPLIT


def kernel(tokens, cb0, cb1, cb2, Wq, bq, Wk, bk, Wv, bv, Wo, bo, gamma, beta):
    t0 = tokens[:, 0]
    t1 = tokens[:, 1]
    t2 = tokens[:, 2]

    wqkv16 = jnp.concatenate([Wq.T, Wk.T, Wv.T], axis=1).astype(jnp.bfloat16)
    bqkv = jnp.concatenate([bq, bk, bv]).reshape(1, 3 * E)
    head_of = jnp.arange(E, dtype=jnp.int32) // DH
    hm = (head_of[:, None] == jnp.arange(H, dtype=jnp.int32)[None, :])
    hm16 = hm.astype(jnp.bfloat16)
    hmt16 = hm.T.astype(jnp.bfloat16)
    wo16 = Wo.T.astype(jnp.bfloat16)
    bo2 = bo.reshape(1, E)
    gamma2 = gamma.reshape(1, E)
    beta2 = beta.reshape(1, E)

    feats = []
    for c in range(NSPLIT):
        sl = slice(c * CHUNK, (c + 1) * CHUNK)
        g = _sc_gather(t0[sl], t1[sl], t2[sl], cb0, cb1, cb2, CHUNK)
        feats.append(_tc_compute(g, wqkv16, bqkv, hm16, hmt16, wo16,
                                 bo2, gamma2, beta2))
    return jnp.concatenate(feats, axis=0)
